# baseline (speedup 1.0000x reference)
import jax
import jax.numpy as jnp
from jax import lax
from jax.experimental import pallas as pl
from jax.experimental.pallas import tpu as pltpu
from jax.experimental.pallas import tpu_sc as plsc

NUM_CLASSES = 1000000
HIDDEN = 64
BATCH = 16384

_NC = 2
_NS = 16
_NW = _NC * _NS
_WIN = 256
_NFULL = NUM_CLASSES // _WIN
_WPW = _NFULL // _NW
_EXTRA = _NFULL - _WPW * _NW
_TAIL0 = _NFULL * _WIN
_LPIECE = 2048
_RING = 32
_NBUF = 3


def _body(labels_hbm, tabt_hbm, out_hbm, lab2_v, mb_v, ml_v, wp_v,
          blk2_v, tail_v, ring_v, sem2, seml, semt, semo):
    wid = lax.axis_index("c") * _NS + lax.axis_index("s")
    nwin = _WPW + jnp.where(wid < _EXTRA, 1, 0)
    lo = (wid * _WPW + jnp.minimum(wid, _EXTRA)) * _WIN
    is_last = wid == _NW - 1
    hi = lo + nwin * _WIN + jnp.where(is_last, HIDDEN, 0)
    iota = lax.iota(jnp.int32, 16)

    for b in range(_NBUF):
        @pl.when(nwin > b)
        def _(b=b):
            pltpu.async_copy(
                tabt_hbm.at[:, pl.ds(pl.multiple_of(lo + b * _WIN, 128),
                                     _WIN)],
                blk2_v.at[b], sem2.at[b])

    pltpu.async_copy(labels_hbm.at[pl.ds(0, _LPIECE)],
                     lab2_v.at[pl.ds(0, _LPIECE)], seml)

    def _piece(p, n):
        @pl.when(p + 1 < BATCH // _LPIECE)
        def _():
            pltpu.async_copy(
                labels_hbm.at[pl.ds((p + 1) * _LPIECE, _LPIECE)],
                lab2_v.at[pl.ds(lax.rem(p + 1, 2) * _LPIECE, _LPIECE)],
                seml)

        par = lax.rem(p, 2)
        pltpu.make_async_copy(
            labels_hbm.at[pl.ds(0, _LPIECE)],
            lab2_v.at[pl.ds(par * _LPIECE, _LPIECE)], seml).wait()

        def _grp(g, n):
            lab16 = lab2_v[pl.ds(par * _LPIECE + g * 16, 16)]
            b16 = iota + (p * _LPIECE + g * 16)
            m = (lab16 >= lo) & (lab16 < hi)
            plsc.store_compressed(mb_v.at[pl.ds(n, 16)], b16, mask=m)
            plsc.store_compressed(ml_v.at[pl.ds(n, 16)], lab16, mask=m)
            return n + plsc.all_reduce_population_count(m)[0]

        return lax.fori_loop(0, _LPIECE // 16, _grp, n)

    n = lax.fori_loop(0, BATCH // _LPIECE, _piece, jnp.int32(0))
    ngrp = (n + 15) // 16

    def _extract(src_v, wcnt, e, dr):
        wgrp = (wcnt + 15) // 16

        def _egrp(j, c):
            e0, dr0 = c
            base = j * 16
            p16 = wp_v[pl.ds(base, 16)]
            col16 = lax.rem(p16, 1024)
            b16 = lax.div(p16, 1024)
            mi = ((iota + base) < wcnt).astype(jnp.int32)
            gcnt = plsc.all_reduce_population_count(mi != 0)[0]
            pos16 = e0 + lax.cumsum(mi, axis=0) - mi
            slot16 = lax.rem(pos16, _RING)
            for k in range(16):
                @pl.when(mi[k] != 0)
                def _():
                    c16 = lax.broadcast(col16[k], (16,))
                    s64 = slot16[k] * HIDDEN
                    for r in range(HIDDEN // 16):
                        v = plsc.load_gather(src_v, [iota + r * 16, c16])
                        ring_v[pl.ds(s64 + r * 16, 16)] = v
                    pltpu.async_copy(
                        ring_v.at[pl.ds(s64, HIDDEN)],
                        out_hbm.at[pl.ds(b16[k] * HIDDEN, HIDDEN)],
                        semo)
            e1 = e0 + gcnt
            do_drain = (e1 - dr0) > 16
            @pl.when(do_drain)
            def _():
                pltpu.make_async_copy(
                    out_hbm.at[pl.ds(0, 16 * HIDDEN)],
                    ring_v.at[pl.ds(0, 16 * HIDDEN)], semo).wait()
            dr1 = jnp.where(do_drain, dr0 + 16, dr0)
            return (e1, dr1)

        return lax.fori_loop(0, wgrp, _egrp, (e, dr))

    def _sublist(c0, width):
        def _wgrp(g, wcnt):
            valid = (iota + g * 16) < n
            lab16 = ml_v[pl.ds(g * 16, 16)]
            b16 = mb_v[pl.ds(g * 16, 16)]
            m = valid & (lab16 >= c0) & (lab16 < c0 + width)
            packed = b16 * 1024 + (lab16 - c0)
            plsc.store_compressed(wp_v.at[pl.ds(wcnt, 16)], packed, mask=m)
            return wcnt + plsc.all_reduce_population_count(m)[0]

        return lax.fori_loop(0, ngrp, _wgrp, jnp.int32(0))

    def _win(k, c):
        e, dr = c
        c0 = pl.multiple_of(lo + k * _WIN, 128)
        par = lax.rem(k, _NBUF)

        pltpu.make_async_copy(tabt_hbm.at[:, pl.ds(c0, _WIN)],
                              blk2_v.at[par], sem2.at[par]).wait()
        wcnt = _sublist(c0, _WIN)
        e1, dr1 = _extract(blk2_v.at[par], wcnt, e, dr)

        @pl.when(k + _NBUF < nwin)
        def _():
            c2 = pl.multiple_of(lo + (k + _NBUF) * _WIN, 128)
            pltpu.async_copy(tabt_hbm.at[:, pl.ds(c2, _WIN)],
                             blk2_v.at[par], sem2.at[par])

        return (e1, dr1)

    e, dr = lax.fori_loop(0, nwin, _win, (jnp.int32(0), jnp.int32(0)))

    @pl.when(is_last)
    def _():
        pltpu.async_copy(tabt_hbm.at[:, pl.ds(_TAIL0, HIDDEN)], tail_v,
                         semt).wait()
        wcnt = _sublist(jnp.int32(_TAIL0), HIDDEN)
        e1, dr1 = _extract(tail_v, wcnt, e, dr)
        _drain_rest(out_hbm, ring_v, semo, e1 - dr1)

    @pl.when(jnp.logical_not(is_last))
    def _():
        _drain_rest(out_hbm, ring_v, semo, e - dr)


def _drain_rest(out_hbm, ring_v, semo, rest):
    def _d(_, __):
        pltpu.make_async_copy(out_hbm.at[pl.ds(0, HIDDEN)],
                              ring_v.at[pl.ds(0, HIDDEN)], semo).wait()
        return __

    lax.fori_loop(0, rest, _d, None)


def kernel(labels, train, table):
    del train
    tabt = table.T
    mesh = plsc.VectorSubcoreMesh(core_axis_name="c", subcore_axis_name="s")
    run = pl.kernel(
        _body,
        mesh=mesh,
        out_type=jax.ShapeDtypeStruct((BATCH * HIDDEN,), jnp.float32),
        scratch_types=[
            pltpu.VMEM((2 * _LPIECE,), jnp.int32),
            pltpu.VMEM((BATCH + 16,), jnp.int32),
            pltpu.VMEM((BATCH + 16,), jnp.int32),
            pltpu.VMEM((BATCH + 16,), jnp.int32),
            pltpu.VMEM((_NBUF, HIDDEN, _WIN), jnp.float32),
            pltpu.VMEM((HIDDEN, HIDDEN), jnp.float32),
            pltpu.VMEM((_RING * HIDDEN,), jnp.float32),
            pltpu.SemaphoreType.DMA((_NBUF,)),
            pltpu.SemaphoreType.DMA,
            pltpu.SemaphoreType.DMA,
            pltpu.SemaphoreType.DMA,
        ],
        compiler_params=pltpu.CompilerParams(needs_layout_passes=False),
    )
    flat = run(labels.astype(jnp.int32), tabt)
    return flat.reshape(BATCH, HIDDEN)


# speedup vs baseline: 1.1234x; 1.1234x over previous
import jax
import jax.numpy as jnp
from jax import lax
from jax.experimental import pallas as pl
from jax.experimental.pallas import tpu as pltpu
from jax.experimental.pallas import tpu_sc as plsc

NUM_CLASSES = 1000000
HIDDEN = 64
BATCH = 16384

_NC = 2
_NS = 16
_NW = _NC * _NS
_WIN = 512
_NFULL = NUM_CLASSES // _WIN
_WPW = _NFULL // _NW
_EXTRA = _NFULL - _WPW * _NW
_TAIL0 = _NFULL * _WIN
_LPIECE = 2048
_RING = 32


def _body(labels_hbm, tabt_hbm, out_hbm, lab2_v, mb_v, ml_v, wp_v,
          blk2_v, tail_v, ring_v, sem2, seml, semt, semo):
    wid = lax.axis_index("c") * _NS + lax.axis_index("s")
    nwin = _WPW + jnp.where(wid < _EXTRA, 1, 0)
    lo = (wid * _WPW + jnp.minimum(wid, _EXTRA)) * _WIN
    is_last = wid == _NW - 1
    hi = lo + nwin * _WIN + jnp.where(is_last, HIDDEN, 0)
    iota = lax.iota(jnp.int32, 16)

    def _fire_window(c0, buf, sem):
        for a in range(8):
            pltpu.async_copy(
                tabt_hbm.at[pl.ds(a * 8, 8), pl.ds(c0, _WIN)],
                blk2_v.at[buf].at[pl.ds(a * 8, 8), :], sem)

    _fire_window(pl.multiple_of(lo, 128), 0, sem2.at[0])
    @pl.when(nwin > 1)
    def _():
        _fire_window(pl.multiple_of(lo + _WIN, 128), 1, sem2.at[1])

    pltpu.async_copy(labels_hbm.at[pl.ds(0, _LPIECE)],
                     lab2_v.at[pl.ds(0, _LPIECE)], seml)

    def _piece(p, n):
        @pl.when(p + 1 < BATCH // _LPIECE)
        def _():
            pltpu.async_copy(
                labels_hbm.at[pl.ds((p + 1) * _LPIECE, _LPIECE)],
                lab2_v.at[pl.ds(lax.rem(p + 1, 2) * _LPIECE, _LPIECE)],
                seml)

        par = lax.rem(p, 2)
        pltpu.make_async_copy(
            labels_hbm.at[pl.ds(0, _LPIECE)],
            lab2_v.at[pl.ds(par * _LPIECE, _LPIECE)], seml).wait()

        def _grp(g, n):
            lab16 = lab2_v[pl.ds(par * _LPIECE + g * 16, 16)]
            b16 = iota + (p * _LPIECE + g * 16)
            m = (lab16 >= lo) & (lab16 < hi)
            plsc.store_compressed(mb_v.at[pl.ds(n, 16)], b16, mask=m)
            plsc.store_compressed(ml_v.at[pl.ds(n, 16)], lab16, mask=m)
            return n + plsc.all_reduce_population_count(m)[0]

        return lax.fori_loop(0, _LPIECE // 16, _grp, n)

    n = lax.fori_loop(0, BATCH // _LPIECE, _piece, jnp.int32(0))
    ngrp = (n + 15) // 16

    def _extract(src_v, wcnt, e, dr):
        wgrp = (wcnt + 15) // 16

        def _egrp(j, c):
            e0, dr0 = c
            base = j * 16
            p16 = wp_v[pl.ds(base, 16)]
            col16 = lax.rem(p16, 1024)
            b16 = lax.div(p16, 1024)
            mi = ((iota + base) < wcnt).astype(jnp.int32)
            gcnt = plsc.all_reduce_population_count(mi != 0)[0]
            pos16 = e0 + lax.cumsum(mi, axis=0) - mi
            slot16 = lax.rem(pos16, _RING)
            for k in range(16):
                @pl.when(mi[k] != 0)
                def _():
                    c16 = lax.broadcast(col16[k], (16,))
                    s64 = slot16[k] * HIDDEN
                    for r in range(HIDDEN // 16):
                        v = plsc.load_gather(src_v, [iota + r * 16, c16])
                        ring_v[pl.ds(s64 + r * 16, 16)] = v
                    pltpu.async_copy(
                        ring_v.at[pl.ds(s64, HIDDEN)],
                        out_hbm.at[pl.ds(b16[k] * HIDDEN, HIDDEN)],
                        semo)
            e1 = e0 + gcnt
            do_drain = (e1 - dr0) > 16
            @pl.when(do_drain)
            def _():
                pltpu.make_async_copy(
                    out_hbm.at[pl.ds(0, 16 * HIDDEN)],
                    ring_v.at[pl.ds(0, 16 * HIDDEN)], semo).wait()
            dr1 = jnp.where(do_drain, dr0 + 16, dr0)
            return (e1, dr1)

        return lax.fori_loop(0, wgrp, _egrp, (e, dr))

    def _sublist(c0, width):
        def _wgrp(g, wcnt):
            valid = (iota + g * 16) < n
            lab16 = ml_v[pl.ds(g * 16, 16)]
            b16 = mb_v[pl.ds(g * 16, 16)]
            m = valid & (lab16 >= c0) & (lab16 < c0 + width)
            packed = b16 * 1024 + (lab16 - c0)
            plsc.store_compressed(wp_v.at[pl.ds(wcnt, 16)], packed, mask=m)
            return wcnt + plsc.all_reduce_population_count(m)[0]

        return lax.fori_loop(0, ngrp, _wgrp, jnp.int32(0))

    def _win(k, c):
        e, dr = c
        c0 = pl.multiple_of(lo + k * _WIN, 128)
        par = lax.rem(k, 2)

        pltpu.make_async_copy(tabt_hbm.at[:, pl.ds(c0, _WIN)],
                              blk2_v.at[par], sem2.at[par]).wait()
        wcnt = _sublist(c0, _WIN)
        e1, dr1 = _extract(blk2_v.at[par], wcnt, e, dr)

        @pl.when(k + 2 < nwin)
        def _():
            c2 = pl.multiple_of(lo + (k + 2) * _WIN, 128)
            _fire_window(c2, par, sem2.at[par])

        return (e1, dr1)

    e, dr = lax.fori_loop(0, nwin, _win, (jnp.int32(0), jnp.int32(0)))

    @pl.when(is_last)
    def _():
        pltpu.async_copy(tabt_hbm.at[:, pl.ds(_TAIL0, HIDDEN)], tail_v,
                         semt).wait()
        wcnt = _sublist(jnp.int32(_TAIL0), HIDDEN)
        e1, dr1 = _extract(tail_v, wcnt, e, dr)
        _drain_rest(out_hbm, ring_v, semo, e1 - dr1)

    @pl.when(jnp.logical_not(is_last))
    def _():
        _drain_rest(out_hbm, ring_v, semo, e - dr)


def _drain_rest(out_hbm, ring_v, semo, rest):
    def _d(_, __):
        pltpu.make_async_copy(out_hbm.at[pl.ds(0, HIDDEN)],
                              ring_v.at[pl.ds(0, HIDDEN)], semo).wait()
        return __

    lax.fori_loop(0, rest, _d, None)


def kernel(labels, train, table):
    del train
    tabt = table.T
    mesh = plsc.VectorSubcoreMesh(core_axis_name="c", subcore_axis_name="s")
    run = pl.kernel(
        _body,
        mesh=mesh,
        out_type=jax.ShapeDtypeStruct((BATCH * HIDDEN,), jnp.float32),
        scratch_types=[
            pltpu.VMEM((2 * _LPIECE,), jnp.int32),
            pltpu.VMEM((BATCH + 16,), jnp.int32),
            pltpu.VMEM((BATCH + 16,), jnp.int32),
            pltpu.VMEM((BATCH + 16,), jnp.int32),
            pltpu.VMEM((2, HIDDEN, _WIN), jnp.float32),
            pltpu.VMEM((HIDDEN, HIDDEN), jnp.float32),
            pltpu.VMEM((_RING * HIDDEN,), jnp.float32),
            pltpu.SemaphoreType.DMA((2,)),
            pltpu.SemaphoreType.DMA,
            pltpu.SemaphoreType.DMA,
            pltpu.SemaphoreType.DMA,
        ],
        compiler_params=pltpu.CompilerParams(needs_layout_passes=False),
    )
    flat = run(labels.astype(jnp.int32), tabt)
    return flat.reshape(BATCH, HIDDEN)
